# baseline (speedup 1.0000x reference)
import functools

import jax
import jax.numpy as jnp
from jax import lax
from jax.experimental import pallas as pl
from jax.experimental.pallas import tpu as pltpu
from jax.experimental.pallas import tpu_sc as plsc

_B = 128
_D = 768
_LANES = 128
_SUB = _D // _LANES
_POOL = 36
_TOPK = 5
_EPL = 20
_HALF = _EPL // 2
_NSEL = _TOPK * _HALF
_PAD = 56
_NIDX = _SUB * _PAD
_SROWS = _B * _NIDX
_PSUB = _POOL * _EPL * _SUB


def _tc_body(x_ref, k2_ref, k3_ref, rk2_ref, rv2_ref, rk3_ref, rv3_ref,
             loss_ref):
    x = x_ref[...]
    q = x / jnp.maximum(jnp.sqrt(jnp.sum(x * x, axis=1, keepdims=True)),
                        1e-12)
    j = lax.broadcasted_iota(jnp.int32, (_B, _NIDX), 1)
    c6 = j // _PAD
    rr = j % _PAD
    seg = rr // _HALF
    valid = rr < _NSEL
    mod = (rr % _HALF) * _SUB + c6
    pad_idx = j % _PSUB
    iota_p = lax.broadcasted_iota(jnp.int32, (_B, _POOL), 1).astype(
        jnp.float32)
    loss_sum = jnp.float32(0.0)
    for k_ref, rk_ref, rv_ref in ((k2_ref, rk2_ref, rv2_ref),
                                  (k3_ref, rk3_ref, rv3_ref)):
        kk = k_ref[...]
        nk = kk / jnp.maximum(
            jnp.sqrt(jnp.sum(kk * kk, axis=1, keepdims=True)), 1e-12)
        cos = lax.dot_general(q, nk, (((1,), (1,)), ((), ())),
                              preferred_element_type=jnp.float32)
        work = cos
        idxs = []
        for _ in range(_TOPK):
            m = jnp.max(work, axis=1, keepdims=True)
            idx = jnp.min(
                jnp.where(work == m, iota_p, jnp.float32(_POOL)),
                axis=1, keepdims=True)
            idxs.append(idx)
            work = jnp.where(iota_p == idx, -jnp.inf, work)
        colsum = jnp.sum(cos, axis=0, keepdims=True)
        sel = jnp.float32(0.0)
        rows_f = jnp.zeros((_B, _NIDX), jnp.float32)
        for t in range(_TOPK):
            sel = sel + jnp.sum(jnp.where(iota_p == idxs[t], colsum, 0.0))
            rows_f = rows_f + jnp.where(
                seg == t, jnp.broadcast_to(idxs[t], (_B, _NIDX)), 0.0)
        rows_k = rows_f.astype(jnp.int32) * (_EPL * _SUB) + mod
        rk_ref[...] = jnp.where(valid, rows_k, pad_idx)
        rv_ref[...] = jnp.where(valid, rows_k + _HALF * _SUB, pad_idx)
        loss_sum = loss_sum + (1.0 - sel / jnp.float32(_B * _B * _TOPK))
    loss_ref[...] = jnp.full((8, 128), loss_sum / jnp.float32(3.0),
                             jnp.float32)


def _tc_select(x, k2, k3):
    return pl.pallas_call(
        _tc_body,
        out_shape=(
            jax.ShapeDtypeStruct((_B, _NIDX), jnp.int32),
            jax.ShapeDtypeStruct((_B, _NIDX), jnp.int32),
            jax.ShapeDtypeStruct((_B, _NIDX), jnp.int32),
            jax.ShapeDtypeStruct((_B, _NIDX), jnp.int32),
            jax.ShapeDtypeStruct((8, 128), jnp.float32),
        ),
    )(x, k2, k3)


def _sc_gather(pool, rk, rv):
    info = plsc.get_sparse_core_info()
    nw = info.num_cores * info.num_subcores
    rpw = _SROWS // nw
    chs = _NIDX
    nch = rpw // chs
    out_t = jax.ShapeDtypeStruct((_SROWS, _LANES), jnp.float32)
    mesh = plsc.VectorSubcoreMesh(core_axis_name="c", subcore_axis_name="s")

    @functools.partial(
        pl.kernel,
        mesh=mesh,
        out_type=[out_t, out_t],
        scratch_types=[
            pltpu.VMEM((chs,), jnp.int32),
            pltpu.VMEM((chs,), jnp.int32),
            pltpu.VMEM((2, chs, _LANES), jnp.float32),
            pltpu.SemaphoreType.DMA,
            pltpu.SemaphoreType.DMA,
            pltpu.SemaphoreType.DMA,
            pltpu.SemaphoreType.DMA,
        ],
    )
    def k(pool_h, rk_h, rv_h, ok, ov, idx_a, idx_b, bufs, g0, g1, w0, w1):
        idxs = (idx_a, idx_b)
        gsems = (g0, g1)
        wsems = (w0, w1)
        wid = lax.axis_index("s") * info.num_cores + lax.axis_index("c")
        base = wid * rpw
        steps = []
        for rows, out in ((rk_h, ok), (rv_h, ov)):
            for c in range(nch):
                steps.append((rows, out, c))
        n = len(steps)

        def start_gather(s, b):
            rows, _, c = steps[s]
            pltpu.sync_copy(rows.at[pl.ds(base + c * chs, chs)], idxs[b])
            return pltpu.async_copy(pool_h.at[idxs[b]], bufs.at[b],
                                    gsems[b])

        def start_write(s, b):
            _, out, c = steps[s]
            return pltpu.async_copy(
                bufs.at[b], out.at[pl.ds(base + c * chs, chs)], wsems[b])

        g = [None, None]
        w = [None, None]
        g[0] = start_gather(0, 0)
        for s in range(n):
            b = s % 2
            nb = 1 - b
            if s + 1 < n:
                if w[nb] is not None:
                    w[nb].wait()
                g[nb] = start_gather(s + 1, nb)
            g[b].wait()
            w[b] = start_write(s, b)
        w[(n - 1) % 2].wait()

    return k(pool, rk, rv)


_BPG = 8


def _relayout_body(ik_ref, iv_ref, ok_ref, ov_ref):
    for src, dst in ((ik_ref, ok_ref), (iv_ref, ov_ref)):
        for bb in range(_BPG):
            for c in range(_SUB):
                dst[bb, :, pl.ds(c * _LANES, _LANES)] = src[
                    pl.ds(bb * _NIDX + c * _PAD, _NSEL), :]


def _tc_relayout(gk, gv):
    out_t = jax.ShapeDtypeStruct((_B, _NSEL, _D), jnp.float32)
    spec_in = pl.BlockSpec((_BPG * _NIDX, _LANES), lambda b: (b, 0))
    spec_out = pl.BlockSpec((_BPG, _NSEL, _D), lambda b: (b, 0, 0))
    return pl.pallas_call(
        _relayout_body,
        grid=(_B // _BPG,),
        in_specs=[spec_in, spec_in],
        out_specs=[spec_out, spec_out],
        out_shape=[out_t, out_t],
    )(gk, gv)


def kernel(x_querry, g_p_0, g_p_1, e_p_2, e_k_2, e_p_3, e_k_3, e_p_4,
           e_k_4):
    del e_p_4, e_k_4
    rk2, rv2, rk3, rv3, loss2d = _tc_select(x_querry, e_k_2, e_k_3)
    gk2, gv2 = _sc_gather(e_p_2.reshape(_PSUB, _LANES),
                          rk2.reshape(-1), rv2.reshape(-1))
    gk3, gv3 = _sc_gather(e_p_3.reshape(_PSUB, _LANES),
                          rk3.reshape(-1), rv3.reshape(-1))
    ok2, ov2 = _tc_relayout(gk2, gv2)
    ok3, ov3 = _tc_relayout(gk3, gv3)
    half_g = 3
    pk0 = jnp.broadcast_to(g_p_0[None, :half_g, :], (_B, half_g, _D))
    pv0 = jnp.broadcast_to(g_p_0[None, half_g:, :], (_B, half_g, _D))
    pk1 = jnp.broadcast_to(g_p_1[None, :half_g, :], (_B, half_g, _D))
    pv1 = jnp.broadcast_to(g_p_1[None, half_g:, :], (_B, half_g, _D))
    return (pk0, pv0, pk1, pv1, ok2, ov2, ok3, ov3, loss2d[0, 0])


# speedup vs baseline: 1.1245x; 1.1245x over previous
import functools

import jax
import jax.numpy as jnp
from jax import lax
from jax.experimental import pallas as pl
from jax.experimental.pallas import tpu as pltpu
from jax.experimental.pallas import tpu_sc as plsc

_B = 128
_D = 768
_LANES = 128
_SUB = _D // _LANES
_POOL = 36
_TOPK = 5
_EPL = 20
_HALF = _EPL // 2
_NSEL = _TOPK * _HALF
_PAD = 56
_NIDX = _SUB * _PAD
_SROWS = _B * _NIDX
_PSUB = _POOL * _EPL * _SUB


def _tc_body(x_ref, k2_ref, k3_ref, rk2_ref, rv2_ref, rk3_ref, rv3_ref,
             loss_ref):
    x = x_ref[...]
    q = x / jnp.maximum(jnp.sqrt(jnp.sum(x * x, axis=1, keepdims=True)),
                        1e-12)
    j = lax.broadcasted_iota(jnp.int32, (_B, _NIDX), 1)
    c6 = j // _PAD
    rr = j % _PAD
    seg = rr // _HALF
    valid = rr < _NSEL
    mod = (rr % _HALF) * _SUB + c6
    bi = lax.broadcasted_iota(jnp.int32, (_B, _NIDX), 0)
    pad_idx = (j + bi * 97) % _PSUB
    iota_p = lax.broadcasted_iota(jnp.int32, (_B, _POOL), 1).astype(
        jnp.float32)
    loss_sum = jnp.float32(0.0)
    for k_ref, rk_ref, rv_ref in ((k2_ref, rk2_ref, rv2_ref),
                                  (k3_ref, rk3_ref, rv3_ref)):
        kk = k_ref[...]
        nk = kk / jnp.maximum(
            jnp.sqrt(jnp.sum(kk * kk, axis=1, keepdims=True)), 1e-12)
        cos = lax.dot_general(q, nk, (((1,), (1,)), ((), ())),
                              preferred_element_type=jnp.float32)
        work = cos
        idxs = []
        for _ in range(_TOPK):
            m = jnp.max(work, axis=1, keepdims=True)
            idx = jnp.min(
                jnp.where(work == m, iota_p, jnp.float32(_POOL)),
                axis=1, keepdims=True)
            idxs.append(idx)
            work = jnp.where(iota_p == idx, -jnp.inf, work)
        colsum = jnp.sum(cos, axis=0, keepdims=True)
        sel = jnp.float32(0.0)
        rows_f = jnp.zeros((_B, _NIDX), jnp.float32)
        for t in range(_TOPK):
            sel = sel + jnp.sum(jnp.where(iota_p == idxs[t], colsum, 0.0))
            rows_f = rows_f + jnp.where(
                seg == t, jnp.broadcast_to(idxs[t], (_B, _NIDX)), 0.0)
        rows_k = rows_f.astype(jnp.int32) * (_EPL * _SUB) + mod
        rk_ref[...] = jnp.where(valid, rows_k, pad_idx)
        rv_ref[...] = jnp.where(valid, rows_k + _HALF * _SUB, pad_idx)
        loss_sum = loss_sum + (1.0 - sel / jnp.float32(_B * _B * _TOPK))
    loss_ref[...] = jnp.full((8, 128), loss_sum / jnp.float32(3.0),
                             jnp.float32)


def _tc_select(x, k2, k3):
    return pl.pallas_call(
        _tc_body,
        out_shape=(
            jax.ShapeDtypeStruct((_B, _NIDX), jnp.int32),
            jax.ShapeDtypeStruct((_B, _NIDX), jnp.int32),
            jax.ShapeDtypeStruct((_B, _NIDX), jnp.int32),
            jax.ShapeDtypeStruct((_B, _NIDX), jnp.int32),
            jax.ShapeDtypeStruct((8, 128), jnp.float32),
        ),
    )(x, k2, k3)


_NB = 4


def _sc_gather(pool2, pool3, rk2, rv2, rk3, rv3):
    info = plsc.get_sparse_core_info()
    nw = info.num_cores * info.num_subcores
    rpw = _SROWS // nw
    chs = rpw // 8
    nch = rpw // chs
    out_t = jax.ShapeDtypeStruct((_SROWS, _LANES), jnp.float32)
    mesh = plsc.VectorSubcoreMesh(core_axis_name="c", subcore_axis_name="s")

    @functools.partial(
        pl.kernel,
        mesh=mesh,
        out_type=[out_t] * 4,
        scratch_types=(
            [pltpu.VMEM((chs,), jnp.int32)] * _NB
            + [pltpu.VMEM((_NB, chs, _LANES), jnp.float32)]
            + [pltpu.SemaphoreType.DMA] * (2 * _NB)
        ),
    )
    def k(p2_h, p3_h, rk2_h, rv2_h, rk3_h, rv3_h, ok2, ov2, ok3, ov3,
          *scr):
        idxs = scr[:_NB]
        bufs = scr[_NB]
        gsems = scr[_NB + 1:2 * _NB + 1]
        wsems = scr[2 * _NB + 1:]
        wid = lax.axis_index("s") * info.num_cores + lax.axis_index("c")
        base = wid * rpw
        steps = []
        for pool, rows, out in ((p2_h, rk2_h, ok2), (p2_h, rv2_h, ov2),
                                (p3_h, rk3_h, ok3), (p3_h, rv3_h, ov3)):
            for c in range(nch):
                steps.append((pool, rows, out, c))
        n = len(steps)

        def start_gather(s, b):
            pool, rows, _, c = steps[s]
            pltpu.sync_copy(rows.at[pl.ds(base + c * chs, chs)], idxs[b])
            return pltpu.async_copy(pool.at[idxs[b]], bufs.at[b],
                                    gsems[b])

        def start_write(s, b):
            _, _, out, c = steps[s]
            return pltpu.async_copy(
                bufs.at[b], out.at[pl.ds(base + c * chs, chs)], wsems[b])

        g = [None] * _NB
        w = [None] * _NB
        for s in range(n):
            b = s % _NB
            if w[b] is not None:
                w[b].wait()
                w[b] = None
            g[b] = start_gather(s, b)
            d = s - (_NB - 1)
            if d >= 0:
                bd = d % _NB
                g[bd].wait()
                w[bd] = start_write(d, bd)
        for d in range(max(0, n - (_NB - 1)), n):
            bd = d % _NB
            g[bd].wait()
            w[bd] = start_write(d, bd)
        for b in range(_NB):
            if w[b] is not None:
                w[b].wait()

    return k(pool2, pool3, rk2, rv2, rk3, rv3)


_BPG = 8


def _relayout_body(ik2_ref, iv2_ref, ik3_ref, iv3_ref,
                   ok2_ref, ov2_ref, ok3_ref, ov3_ref):
    for src, dst in ((ik2_ref, ok2_ref), (iv2_ref, ov2_ref),
                     (ik3_ref, ok3_ref), (iv3_ref, ov3_ref)):
        for bb in range(_BPG):
            for c in range(_SUB):
                dst[bb, :, pl.ds(c * _LANES, _LANES)] = src[
                    pl.ds(bb * _NIDX + c * _PAD, _NSEL), :]


def _tc_relayout(gk2, gv2, gk3, gv3):
    out_t = jax.ShapeDtypeStruct((_B, _NSEL, _D), jnp.float32)
    spec_in = pl.BlockSpec((_BPG * _NIDX, _LANES), lambda b: (b, 0))
    spec_out = pl.BlockSpec((_BPG, _NSEL, _D), lambda b: (b, 0, 0))
    return pl.pallas_call(
        _relayout_body,
        grid=(_B // _BPG,),
        in_specs=[spec_in] * 4,
        out_specs=[spec_out] * 4,
        out_shape=[out_t] * 4,
    )(gk2, gv2, gk3, gv3)


def kernel(x_querry, g_p_0, g_p_1, e_p_2, e_k_2, e_p_3, e_k_3, e_p_4,
           e_k_4):
    del e_p_4, e_k_4
    rk2, rv2, rk3, rv3, loss2d = _tc_select(x_querry, e_k_2, e_k_3)
    gk2, gv2, gk3, gv3 = _sc_gather(
        e_p_2.reshape(_PSUB, _LANES), e_p_3.reshape(_PSUB, _LANES),
        rk2.reshape(-1), rv2.reshape(-1), rk3.reshape(-1), rv3.reshape(-1))
    ok2, ov2, ok3, ov3 = _tc_relayout(gk2, gv2, gk3, gv3)
    half_g = 3
    pk0 = jnp.broadcast_to(g_p_0[None, :half_g, :], (_B, half_g, _D))
    pv0 = jnp.broadcast_to(g_p_0[None, half_g:, :], (_B, half_g, _D))
    pk1 = jnp.broadcast_to(g_p_1[None, :half_g, :], (_B, half_g, _D))
    pv1 = jnp.broadcast_to(g_p_1[None, half_g:, :], (_B, half_g, _D))
    return (pk0, pv0, pk1, pv1, ok2, ov2, ok3, ov3, loss2d[0, 0])
